# baseline (speedup 1.0000x reference)
import jax
import jax.numpy as jnp
from jax.experimental import pallas as pl

_NUM_GRAPHS = 256


def _mm_kernel(x_ref, w_ref, o_ref):
    o_ref[...] = jnp.dot(x_ref[...], w_ref[...],
                         preferred_element_type=jnp.float32)


def _pallas_matmul(x, w, block_rows=2048):
    n, k = x.shape
    m = w.shape[1]
    pad = (-n) % block_rows
    if pad:
        x = jnp.pad(x, ((0, pad), (0, 0)))
    npad = x.shape[0]
    out = pl.pallas_call(
        _mm_kernel,
        grid=(npad // block_rows,),
        in_specs=[pl.BlockSpec((block_rows, k), lambda i: (i, 0)),
                  pl.BlockSpec((k, m), lambda i: (0, 0))],
        out_specs=pl.BlockSpec((block_rows, m), lambda i: (i, 0)),
        out_shape=jax.ShapeDtypeStruct((npad, m), jnp.float32),
    )(x, w)
    return out[:n]


def _head_kernel(mp_ref, tp_ref,
                 wm1_ref, bm1_ref, wm2_ref, bm2_ref,
                 wp1_ref, bp1_ref, wp2_ref, bp2_ref,
                 wf1_ref, bf1_ref, wf2_ref, bf2_ref,
                 wo_ref, bo_ref, o_ref):
    dot = lambda a, b: jnp.dot(a, b, preferred_element_type=jnp.float32)
    relu = lambda v: jnp.maximum(v, 0.0)
    mx = relu(dot(mp_ref[...], wm1_ref[...]) + bm1_ref[...])
    mx = dot(mx, wm2_ref[...]) + bm2_ref[...]
    tx = relu(dot(tp_ref[...], wp1_ref[...]) + bp1_ref[...])
    tx = dot(tx, wp2_ref[...]) + bp2_ref[...]
    xc = jnp.concatenate([mx, tx], axis=1)
    xc = relu(dot(xc, wf1_ref[...]) + bf1_ref[...])
    xc = relu(dot(xc, wf2_ref[...]) + bf2_ref[...])
    o_ref[...] = dot(xc, wo_ref[...]) + bo_ref[...]


def _mlp_head(mp, tp, W_mfg1, b_mfg1, W_mfg2, b_mfg2,
              W_pfg1, b_pfg1, W_pfg2, b_pfg2,
              W_fc1, b_fc1, W_fc2, b_fc2, W_out, b_out):
    r2 = lambda b: b.reshape(1, -1)
    args = (mp, tp,
            W_mfg1, r2(b_mfg1), W_mfg2, r2(b_mfg2),
            W_pfg1, r2(b_pfg1), W_pfg2, r2(b_pfg2),
            W_fc1, r2(b_fc1), W_fc2, r2(b_fc2),
            W_out, r2(b_out))
    return pl.pallas_call(
        _head_kernel,
        out_shape=jax.ShapeDtypeStruct((_NUM_GRAPHS, W_out.shape[1]),
                                       jnp.float32),
    )(*args)


def _gat_conv(x, edge_index, W, a_src, a_dst, b, heads, out_ch):
    N = x.shape[0]
    loop = jnp.arange(N, dtype=edge_index.dtype)
    src = jnp.concatenate([edge_index[0], loop])
    dst = jnp.concatenate([edge_index[1], loop])
    xp = _pallas_matmul(x, W).reshape(N, heads, out_ch)
    a_s = jnp.sum(xp * a_src[None, :, :], axis=-1)
    a_d = jnp.sum(xp * a_dst[None, :, :], axis=-1)
    e = jax.nn.leaky_relu(a_s[src] + a_d[dst], negative_slope=0.2)
    m = jax.ops.segment_max(e, dst, num_segments=N)
    m = jnp.where(jnp.isfinite(m), m, 0.0)
    ex = jnp.exp(e - m[dst])
    s = jax.ops.segment_sum(ex, dst, num_segments=N)
    alpha = ex / (s[dst] + 1e-16)
    out = jax.ops.segment_sum(xp[src] * alpha[:, :, None], dst,
                              num_segments=N)
    return out.reshape(N, heads * out_ch) + b


def _gcn_conv(x, edge_index, W, b):
    N = x.shape[0]
    loop = jnp.arange(N, dtype=edge_index.dtype)
    src = jnp.concatenate([edge_index[0], loop])
    dst = jnp.concatenate([edge_index[1], loop])
    deg = jax.ops.segment_sum(jnp.ones(src.shape[0], x.dtype), dst,
                              num_segments=N)
    dinv = jnp.where(deg > 0, jax.lax.rsqrt(jnp.maximum(deg, 1e-12)), 0.0)
    xw = _pallas_matmul(x, W)
    msg = xw[src] * (dinv[src] * dinv[dst])[:, None]
    return jax.ops.segment_sum(msg, dst, num_segments=N) + b


def _global_pool(x, batch):
    s = jax.ops.segment_sum(x, batch, num_segments=_NUM_GRAPHS)
    cnt = jax.ops.segment_sum(jnp.ones(x.shape[0], x.dtype), batch,
                              num_segments=_NUM_GRAPHS)
    mean = s / jnp.maximum(cnt, 1.0)[:, None]
    mx = jax.ops.segment_max(x, batch, num_segments=_NUM_GRAPHS)
    mx = jnp.where(cnt[:, None] > 0, mx, 0.0)
    return jnp.concatenate([mx, mean], axis=1)


def kernel(mol_x, target_x, W_mg, a_src_m, a_dst_m, b_mg, W_mc, b_mc,
           W_mfg1, b_mfg1, W_mfg2, b_mfg2, W_pg, a_src_p, a_dst_p, b_pg,
           W_pc, b_pc, W_pfg1, b_pfg1, W_pfg2, b_pfg2, W_fc1, b_fc1,
           W_fc2, b_fc2, W_out, b_out, mol_edge_index, mol_batch,
           target_edge_index, target_batch):
    relu = jax.nn.relu
    mx = relu(_gat_conv(mol_x, mol_edge_index, W_mg, a_src_m, a_dst_m,
                        b_mg, 2, 78))
    mx = relu(_gcn_conv(mx, mol_edge_index, W_mc, b_mc))
    mp = _global_pool(mx, mol_batch)
    tx = relu(_gat_conv(target_x, target_edge_index, W_pg, a_src_p,
                        a_dst_p, b_pg, 2, 54))
    tx = relu(_gcn_conv(tx, target_edge_index, W_pc, b_pc))
    tp = _global_pool(tx, target_batch)
    return _mlp_head(mp, tp, W_mfg1, b_mfg1, W_mfg2, b_mfg2,
                     W_pfg1, b_pfg1, W_pfg2, b_pfg2,
                     W_fc1, b_fc1, W_fc2, b_fc2, W_out, b_out)
